# baseline (speedup 1.0000x reference)
import functools

import jax
import jax.numpy as jnp
import numpy as np
from jax import lax
from jax.experimental import pallas as pl
from jax.experimental.pallas import tpu as pltpu
from jax.experimental.pallas import tpu_sc as plsc

_VOCAB = 29
_D = 1024
_SEQ = 2000
_BATCH = 4
_ROWS = _BATCH * _SEQ
_NC = 2
_NS = 16
_NW = _NC * _NS
_RPW = _ROWS // _NW
_CHUNK = 25
_NCHUNK = _RPW // _CHUNK
_SUB_PER_BATCH = _SEQ // _RPW
_NV = _D // 16


def _pos_encoding() -> np.ndarray:
    pos = np.arange(_SEQ, dtype=np.float32).reshape(-1, 1)
    div = np.power(10000.0, np.arange(0, _D, 2, dtype=np.float32) / _D)
    x = pos / div
    p = np.zeros((_SEQ, _D), dtype=np.float32)
    p[:, 0::2] = np.sin(x)
    p[:, 1::2] = np.cos(x)
    return p


_P_CONST = _pos_encoding()


def _sc_body(idx_hbm, p_hbm, table_hbm, g_hbm, b_hbm, out_hbm,
             idx_v, t_v, p_v, g_v, b_v, sem):
    w = lax.axis_index("s") * _NC + lax.axis_index("c")
    pltpu.sync_copy(g_hbm, g_v)
    pltpu.sync_copy(b_hbm, b_v)

    def chunk_body(c, carry):
        r0 = w * _RPW + c * _CHUNK
        l0 = (w % _SUB_PER_BATCH) * _RPW + c * _CHUNK
        pltpu.sync_copy(idx_hbm.at[w * _NCHUNK + c], idx_v)
        pltpu.sync_copy(p_hbm.at[pl.ds(l0, _CHUNK), :], p_v)
        pltpu.async_copy(table_hbm.at[idx_v], t_v, sem).wait()

        def row_body(r, rcarry):
            zero = jnp.zeros((16,), jnp.float32)

            def pass1(j, acc):
                s, s2 = acc
                e = t_v[r, pl.ds(j * 16, 16)] + p_v[r, pl.ds(j * 16, 16)]
                p_v[r, pl.ds(j * 16, 16)] = e
                return (s + e, s2 + e * e)

            s, s2 = lax.fori_loop(0, _NV, pass1, (zero, zero))
            tot = jnp.sum(s)
            tot2 = jnp.sum(s2)
            mean = tot * (1.0 / _D)
            var = tot2 * (1.0 / _D) - mean * mean
            x = var + 1e-12
            xi = lax.bitcast_convert_type(x, jnp.int32)
            y = lax.bitcast_convert_type(
                jnp.int32(0x5F3759DF) - (xi >> 1), jnp.float32)
            y = y * (1.5 - 0.5 * x * y * y)
            y = y * (1.5 - 0.5 * x * y * y)
            y = y * (1.5 - 0.5 * x * y * y)
            m_s = jnp.full((16,), mean, jnp.float32)
            y_s = jnp.full((16,), y, jnp.float32)

            def pass2(j, _):
                e = p_v[r, pl.ds(j * 16, 16)]
                o = ((e - m_s) * y_s * g_v[pl.ds(j * 16, 16)]
                     + b_v[pl.ds(j * 16, 16)])
                p_v[r, pl.ds(j * 16, 16)] = o
                return 0

            lax.fori_loop(0, _NV, pass2, 0)
            return rcarry

        lax.fori_loop(0, _CHUNK, row_body, 0)
        pltpu.sync_copy(p_v, out_hbm.at[pl.ds(r0, _CHUNK), :])
        return carry

    lax.fori_loop(0, _NCHUNK, chunk_body, 0)


@functools.partial(jax.jit, static_argnames=())
def kernel(batch, table, gamma, beta):
    idx = batch.astype(jnp.int32).reshape(_NW * _NCHUNK, _CHUNK)
    p = jnp.asarray(_P_CONST)
    mesh = plsc.VectorSubcoreMesh(core_axis_name="c", subcore_axis_name="s",
                                  num_cores=_NC, num_subcores=_NS)
    run = pl.kernel(
        _sc_body,
        out_type=jax.ShapeDtypeStruct((_ROWS, _D), jnp.float32),
        mesh=mesh,
        scratch_types=[
            pltpu.VMEM((_CHUNK,), jnp.int32),
            pltpu.VMEM((_CHUNK, _D), jnp.float32),
            pltpu.VMEM((_CHUNK, _D), jnp.float32),
            pltpu.VMEM((_D,), jnp.float32),
            pltpu.VMEM((_D,), jnp.float32),
            pltpu.SemaphoreType.DMA,
        ],
        compiler_params=pltpu.CompilerParams(use_tc_tiling_on_sc=False,
                                             needs_layout_passes=False),
    )
    out = run(idx, p, table.astype(jnp.float32), gamma.astype(jnp.float32),
              beta.astype(jnp.float32))
    return out.reshape(_BATCH, _SEQ, _D)


# speedup vs baseline: 1.5235x; 1.5235x over previous
import functools

import jax
import jax.numpy as jnp
import numpy as np
from jax import lax
from jax.experimental import pallas as pl
from jax.experimental.pallas import tpu as pltpu
from jax.experimental.pallas import tpu_sc as plsc

_VOCAB = 29
_D = 1024
_SEQ = 2000
_BATCH = 4
_ROWS = _BATCH * _SEQ
_NC = 2
_NS = 16
_NW = _NC * _NS
_RPW = _ROWS // _NW
_CHUNK = 25
_NCHUNK = _RPW // _CHUNK
_SUB_PER_BATCH = _SEQ // _RPW
_NV = _D // 16


def _pos_encoding() -> np.ndarray:
    pos = np.arange(_SEQ, dtype=np.float32).reshape(-1, 1)
    div = np.power(10000.0, np.arange(0, _D, 2, dtype=np.float32) / _D)
    x = pos / div
    p = np.zeros((_SEQ, _D), dtype=np.float32)
    p[:, 0::2] = np.sin(x)
    p[:, 1::2] = np.cos(x)
    return p


_P_CONST = _pos_encoding()


def _sc_body(idx_hbm, p_hbm, table_hbm, g_hbm, b_hbm, out_hbm,
             idx_v, t_v, p_v, g_v, b_v, sem):
    w = lax.axis_index("s") * _NC + lax.axis_index("c")
    pltpu.sync_copy(g_hbm, g_v)
    pltpu.sync_copy(b_hbm, b_v)

    def chunk_body(c, carry):
        r0 = w * _RPW + c * _CHUNK
        l0 = (w % _SUB_PER_BATCH) * _RPW + c * _CHUNK
        pltpu.sync_copy(idx_hbm.at[w * _NCHUNK + c], idx_v)
        pltpu.sync_copy(p_hbm.at[pl.ds(l0, _CHUNK), :], p_v)
        pltpu.async_copy(table_hbm.at[idx_v], t_v, sem).wait()

        def row_body(r, rcarry):
            zero = jnp.zeros((16,), jnp.float32)

            s = [zero] * 4
            s2 = [zero] * 4
            for j in range(_NV):
                e = t_v[r, pl.ds(j * 16, 16)] + p_v[r, pl.ds(j * 16, 16)]
                p_v[r, pl.ds(j * 16, 16)] = e
                s[j % 4] = s[j % 4] + e
                s2[j % 4] = s2[j % 4] + e * e
            tot = jnp.sum((s[0] + s[1]) + (s[2] + s[3]))
            tot2 = jnp.sum((s2[0] + s2[1]) + (s2[2] + s2[3]))
            mean = tot * (1.0 / _D)
            var = tot2 * (1.0 / _D) - mean * mean
            x = var + 1e-12
            xi = lax.bitcast_convert_type(x, jnp.int32)
            y = lax.bitcast_convert_type(
                jnp.int32(0x5F3759DF) - (xi >> 1), jnp.float32)
            y = y * (1.5 - 0.5 * x * y * y)
            y = y * (1.5 - 0.5 * x * y * y)
            y = y * (1.5 - 0.5 * x * y * y)
            m_s = jnp.full((16,), mean, jnp.float32)
            y_s = jnp.full((16,), y, jnp.float32)

            for j in range(_NV):
                e = p_v[r, pl.ds(j * 16, 16)]
                o = ((e - m_s) * y_s * g_v[pl.ds(j * 16, 16)]
                     + b_v[pl.ds(j * 16, 16)])
                p_v[r, pl.ds(j * 16, 16)] = o
            return rcarry

        lax.fori_loop(0, _CHUNK, row_body, 0)
        pltpu.sync_copy(p_v, out_hbm.at[pl.ds(r0, _CHUNK), :])
        return carry

    lax.fori_loop(0, _NCHUNK, chunk_body, 0)


@functools.partial(jax.jit, static_argnames=())
def kernel(batch, table, gamma, beta):
    idx = batch.astype(jnp.int32).reshape(_NW * _NCHUNK, _CHUNK)
    p = jnp.asarray(_P_CONST)
    mesh = plsc.VectorSubcoreMesh(core_axis_name="c", subcore_axis_name="s",
                                  num_cores=_NC, num_subcores=_NS)
    run = pl.kernel(
        _sc_body,
        out_type=jax.ShapeDtypeStruct((_ROWS, _D), jnp.float32),
        mesh=mesh,
        scratch_types=[
            pltpu.VMEM((_CHUNK,), jnp.int32),
            pltpu.VMEM((_CHUNK, _D), jnp.float32),
            pltpu.VMEM((_CHUNK, _D), jnp.float32),
            pltpu.VMEM((_D,), jnp.float32),
            pltpu.VMEM((_D,), jnp.float32),
            pltpu.SemaphoreType.DMA,
        ],
        compiler_params=pltpu.CompilerParams(use_tc_tiling_on_sc=False,
                                             needs_layout_passes=False),
    )
    out = run(idx, p, table.astype(jnp.float32), gamma.astype(jnp.float32),
              beta.astype(jnp.float32))
    return out.reshape(_BATCH, _SEQ, _D)


# speedup vs baseline: 3.0318x; 1.9900x over previous
import functools

import jax
import jax.numpy as jnp
import numpy as np
from jax import lax
from jax.experimental import pallas as pl
from jax.experimental.pallas import tpu as pltpu
from jax.experimental.pallas import tpu_sc as plsc

_VOCAB = 29
_VPAD = 32
_D = 1024
_SEQ = 2000
_BATCH = 4
_ROWS = _BATCH * _SEQ
_NC = 2
_NS = 16
_NW = _NC * _NS
_LPW = 64
_LCH = 8
_NCHUNK = _LPW // _LCH
_NV = _D // 16


def _pos_encoding() -> np.ndarray:
    pos = np.arange(_SEQ, dtype=np.float32).reshape(-1, 1)
    div = np.power(10000.0, np.arange(0, _D, 2, dtype=np.float32) / _D)
    x = pos / div
    p = np.zeros((_SEQ, _D), dtype=np.float32)
    p[:, 0::2] = np.sin(x)
    p[:, 1::2] = np.cos(x)
    return p


_P_CONST = _pos_encoding()


def _tc_stats(batch_ref, table_ref, p_ref, gamma_ref,
              a_ref, t2_ref, p2_ref):
    tab = table_ref[:]
    pos = p_ref[:]
    g = gamma_ref[:]
    inv_d = jnp.float32(1.0 / _D)
    st = jnp.sum(tab, axis=1)
    qt = jnp.sum(tab * tab, axis=1)
    sp = jnp.sum(pos, axis=1)
    qp = jnp.sum(pos * pos, axis=1)
    ct = lax.dot_general(pos, tab, (((1,), (1,)), ((), ())),
                         precision=lax.Precision.HIGHEST,
                         preferred_element_type=jnp.float32)
    tok = batch_ref[:]
    oh = (tok[:, :, None]
          == lax.broadcasted_iota(jnp.int32, (1, 1, _VPAD), 2)
          ).astype(jnp.float32)
    st_sel = jnp.sum(oh * st[None, None, :], axis=-1)
    qt_sel = jnp.sum(oh * qt[None, None, :], axis=-1)
    c_sel = jnp.sum(oh * ct[None, :, :], axis=-1)
    mean = (st_sel + sp[None, :]) * inv_d
    e2 = (qt_sel + 2.0 * c_sel + qp[None, :]) * inv_d
    var = e2 - mean * mean
    a_ref[:] = lax.rsqrt(var + 1e-12)
    t2_ref[:] = g * (tab - st[:, None] * inv_d)
    p2_ref[:] = g * (pos - sp[:, None] * inv_d)


def _sc_body(idx_hbm, a_hbm, t2_hbm, p2_hbm, beta_hbm, out_hbm,
             t2f_v, beta_v, p2_v, idx_v, a_v, out_v,
             sem_in0, sem_in1, sem_out0, sem_out1):
    w = lax.axis_index("s") * _NC + lax.axis_index("c")
    l_base = jnp.minimum(_LPW * w, _SEQ - _LPW)
    pltpu.sync_copy(t2_hbm, t2f_v)
    pltpu.sync_copy(beta_hbm, beta_v)
    sems_in = (sem_in0, sem_in1)
    sems_out = (sem_out0, sem_out1)
    iota = lax.broadcasted_iota(jnp.int32, (16,), 0)

    def l_of(c):
        return jnp.minimum(l_base + c * _LCH, _SEQ - _LCH)

    def in_copies(c, par):
        l0 = l_of(c)
        return (
            pltpu.make_async_copy(p2_hbm.at[pl.ds(l0, _LCH), :],
                                  p2_v.at[par], sems_in[par]),
            pltpu.make_async_copy(idx_hbm.at[pl.ds(l0, _LCH)],
                                  idx_v.at[par], sems_in[par]),
            pltpu.make_async_copy(a_hbm.at[pl.ds(l0, _LCH)],
                                  a_v.at[par], sems_in[par]),
        )

    def out_copies(c, par):
        l0 = l_of(c)
        return tuple(
            pltpu.make_async_copy(out_v.at[par, b],
                                  out_hbm.at[pl.ds(b * _SEQ + l0, _LCH), :],
                                  sems_out[par])
            for b in range(_BATCH))

    def issue(copies):
        for d in copies:
            d.start()

    def wait(copies):
        for d in copies:
            d.wait()

    def compute(par):
        for lp in range(_LCH):
            toks = [idx_v[par, lp, b, :] for b in range(_BATCH)]
            asp = [a_v[par, lp, b, :] for b in range(_BATCH)]
            bases = [toks[b] * _D + iota for b in range(_BATCH)]

            @plsc.parallel_loop(0, _NV, 1, unroll=4)
            def jbody(i, _par=par, _lp=lp, _bases=bases, _asp=asp):
                joff = i * 16
                bq = beta_v[pl.ds(joff, 16)]
                pq = p2_v[_par, _lp, pl.ds(joff, 16)]
                for b in range(_BATCH):
                    t2 = plsc.load_gather(t2f_v, [_bases[b] + joff])
                    o = (t2 + pq) * _asp[b] + bq
                    out_v[_par, b, _lp, pl.ds(joff, 16)] = o

    def chunk_body(c, carry):
        issue(in_copies(c, 0))
        wait(in_copies(c, 0))
        compute(0)
        issue(out_copies(c, 0))
        wait(out_copies(c, 0))
        return carry

    lax.fori_loop(0, _NCHUNK, chunk_body, 0)


@jax.jit
def kernel(batch, table, gamma, beta):
    batch = batch.astype(jnp.int32)
    table = table.astype(jnp.float32)
    gamma = gamma.astype(jnp.float32)
    beta = beta.astype(jnp.float32)
    p = jnp.asarray(_P_CONST)
    tab_pad = jnp.pad(table, ((0, _VPAD - _VOCAB), (0, 0)))

    a, t2, p2 = pl.pallas_call(
        _tc_stats,
        out_shape=[
            jax.ShapeDtypeStruct((_BATCH, _SEQ), jnp.float32),
            jax.ShapeDtypeStruct((_VPAD, _D), jnp.float32),
            jax.ShapeDtypeStruct((_SEQ, _D), jnp.float32),
        ],
    )(batch, tab_pad, p, gamma.reshape(1, _D))

    idx_t = jnp.broadcast_to(batch.T.reshape(_SEQ, _BATCH, 1),
                             (_SEQ, _BATCH, 16))
    a_t = jnp.broadcast_to(a.T.reshape(_SEQ, _BATCH, 1),
                           (_SEQ, _BATCH, 16))
    t2f = t2.reshape(_VPAD * _D)

    mesh = plsc.VectorSubcoreMesh(core_axis_name="c", subcore_axis_name="s",
                                  num_cores=_NC, num_subcores=_NS)
    run = pl.kernel(
        _sc_body,
        out_type=jax.ShapeDtypeStruct((_ROWS, _D), jnp.float32),
        mesh=mesh,
        scratch_types=[
            pltpu.VMEM((_VPAD * _D,), jnp.float32),
            pltpu.VMEM((_D,), jnp.float32),
            pltpu.VMEM((2, _LCH, _D), jnp.float32),
            pltpu.VMEM((2, _LCH, _BATCH, 16), jnp.int32),
            pltpu.VMEM((2, _LCH, _BATCH, 16), jnp.float32),
            pltpu.VMEM((2, _BATCH, _LCH, _D), jnp.float32),
            pltpu.SemaphoreType.DMA,
            pltpu.SemaphoreType.DMA,
            pltpu.SemaphoreType.DMA,
            pltpu.SemaphoreType.DMA,
        ],
        compiler_params=pltpu.CompilerParams(use_tc_tiling_on_sc=False,
                                             needs_layout_passes=False),
    )
    out = run(idx_t, a_t, t2f, p2, beta)
    return out.reshape(_BATCH, _SEQ, _D)


# speedup vs baseline: 3.2238x; 1.0633x over previous
import functools

import jax
import jax.numpy as jnp
import numpy as np
from jax import lax
from jax.experimental import pallas as pl
from jax.experimental.pallas import tpu as pltpu
from jax.experimental.pallas import tpu_sc as plsc

_VOCAB = 29
_VPAD = 32
_D = 1024
_SEQ = 2000
_BATCH = 4
_ROWS = _BATCH * _SEQ
_NC = 2
_NS = 16
_NW = _NC * _NS
_LPW = 64
_LCH = 8
_NCHUNK = _LPW // _LCH
_NV = _D // 16


def _pos_encoding() -> np.ndarray:
    pos = np.arange(_SEQ, dtype=np.float32).reshape(-1, 1)
    div = np.power(10000.0, np.arange(0, _D, 2, dtype=np.float32) / _D)
    x = pos / div
    p = np.zeros((_SEQ, _D), dtype=np.float32)
    p[:, 0::2] = np.sin(x)
    p[:, 1::2] = np.cos(x)
    return p


_P_CONST = _pos_encoding()


def _tc_stats(batch_ref, table_ref, p_ref, gamma_ref,
              a_ref, t2_ref, p2_ref):
    tab = table_ref[:]
    pos = p_ref[:]
    g = gamma_ref[:]
    inv_d = jnp.float32(1.0 / _D)
    st = jnp.sum(tab, axis=1)
    qt = jnp.sum(tab * tab, axis=1)
    sp = jnp.sum(pos, axis=1)
    qp = jnp.sum(pos * pos, axis=1)
    ct = lax.dot_general(pos, tab, (((1,), (1,)), ((), ())),
                         precision=lax.Precision.HIGHEST,
                         preferred_element_type=jnp.float32)
    tok = batch_ref[:]
    oh = (tok[:, :, None]
          == lax.broadcasted_iota(jnp.int32, (1, 1, _VPAD), 2)
          ).astype(jnp.float32)
    st_sel = jnp.sum(oh * st[None, None, :], axis=-1)
    qt_sel = jnp.sum(oh * qt[None, None, :], axis=-1)
    c_sel = jnp.sum(oh * ct[None, :, :], axis=-1)
    mean = (st_sel + sp[None, :]) * inv_d
    e2 = (qt_sel + 2.0 * c_sel + qp[None, :]) * inv_d
    var = e2 - mean * mean
    a_ref[:] = lax.rsqrt(var + 1e-12)
    t2_ref[:] = g * (tab - st[:, None] * inv_d)
    p2_ref[:] = g * (pos - sp[:, None] * inv_d)


def _sc_body(idx_hbm, a_hbm, t2_hbm, p2_hbm, beta_hbm, out_hbm,
             t2f_v, beta_v, p2_v, idx_v, a_v, out_v,
             sem_in0, sem_in1, sem_out0, sem_out1):
    w = lax.axis_index("s") * _NC + lax.axis_index("c")
    l_base = jnp.minimum(_LPW * w, _SEQ - _LPW)
    pltpu.sync_copy(t2_hbm, t2f_v)
    pltpu.sync_copy(beta_hbm, beta_v)
    sems_in = (sem_in0, sem_in1)
    sems_out = (sem_out0, sem_out1)
    iota = lax.broadcasted_iota(jnp.int32, (16,), 0)

    def l_of(c):
        return jnp.minimum(l_base + c * _LCH, _SEQ - _LCH)

    def in_copies(c, par):
        l0 = l_of(c)
        return (
            pltpu.make_async_copy(p2_hbm.at[pl.ds(l0, _LCH), :],
                                  p2_v.at[par], sems_in[par]),
            pltpu.make_async_copy(idx_hbm.at[pl.ds(l0, _LCH)],
                                  idx_v.at[par], sems_in[par]),
            pltpu.make_async_copy(a_hbm.at[pl.ds(l0, _LCH)],
                                  a_v.at[par], sems_in[par]),
        )

    def out_copies(c, par):
        l0 = l_of(c)
        return tuple(
            pltpu.make_async_copy(out_v.at[par, b],
                                  out_hbm.at[pl.ds(b * _SEQ + l0, _LCH), :],
                                  sems_out[par])
            for b in range(_BATCH))

    def issue(copies):
        for d in copies:
            d.start()

    def wait(copies):
        for d in copies:
            d.wait()

    def compute(par):
        for lp in range(_LCH):
            toks = [idx_v[par, lp, b, :] for b in range(_BATCH)]
            asp = [a_v[par, lp, b, :] for b in range(_BATCH)]
            bases = [toks[b] * _D + iota for b in range(_BATCH)]

            @plsc.parallel_loop(0, _NV, 1, unroll=4)
            def jbody(i, _par=par, _lp=lp, _bases=bases, _asp=asp):
                joff = i * 16
                bq = beta_v[pl.ds(joff, 16)]
                pq = p2_v[_par, _lp, pl.ds(joff, 16)]
                for b in range(_BATCH):
                    t2 = plsc.load_gather(t2f_v, [_bases[b] + joff])
                    o = (t2 + pq) * _asp[b] + bq
                    out_v[_par, b, _lp, pl.ds(joff, 16)] = o

    issue(in_copies(0, 0))

    def pair_body(i, carry):
        for u in range(2):
            c = 2 * i + u
            issue(in_copies(c + 1, 1 - u))
            wait(in_copies(c, u))
            pl.when(i > 0)(lambda _u=u, _c=c: wait(out_copies(_c - 2, _u)))
            compute(u)
            issue(out_copies(c, u))
        return carry

    lax.fori_loop(0, _NCHUNK // 2, pair_body, 0)

    wait(out_copies(_NCHUNK - 2, 0))
    wait(out_copies(_NCHUNK - 1, 1))
    wait(in_copies(_NCHUNK, 0))


@jax.jit
def kernel(batch, table, gamma, beta):
    batch = batch.astype(jnp.int32)
    table = table.astype(jnp.float32)
    gamma = gamma.astype(jnp.float32)
    beta = beta.astype(jnp.float32)
    p = jnp.asarray(_P_CONST)
    tab_pad = jnp.pad(table, ((0, _VPAD - _VOCAB), (0, 0)))

    a, t2, p2 = pl.pallas_call(
        _tc_stats,
        out_shape=[
            jax.ShapeDtypeStruct((_BATCH, _SEQ), jnp.float32),
            jax.ShapeDtypeStruct((_VPAD, _D), jnp.float32),
            jax.ShapeDtypeStruct((_SEQ, _D), jnp.float32),
        ],
    )(batch, tab_pad, p, gamma.reshape(1, _D))

    idx_t = jnp.broadcast_to(batch.T.reshape(_SEQ, _BATCH, 1),
                             (_SEQ, _BATCH, 16))
    a_t = jnp.broadcast_to(a.T.reshape(_SEQ, _BATCH, 1),
                           (_SEQ, _BATCH, 16))
    t2f = t2.reshape(_VPAD * _D)

    mesh = plsc.VectorSubcoreMesh(core_axis_name="c", subcore_axis_name="s",
                                  num_cores=_NC, num_subcores=_NS)
    run = pl.kernel(
        _sc_body,
        out_type=jax.ShapeDtypeStruct((_ROWS, _D), jnp.float32),
        mesh=mesh,
        scratch_types=[
            pltpu.VMEM((_VPAD * _D,), jnp.float32),
            pltpu.VMEM((_D,), jnp.float32),
            pltpu.VMEM((2, _LCH, _D), jnp.float32),
            pltpu.VMEM((2, _LCH, _BATCH, 16), jnp.int32),
            pltpu.VMEM((2, _LCH, _BATCH, 16), jnp.float32),
            pltpu.VMEM((2, _BATCH, _LCH, _D), jnp.float32),
            pltpu.SemaphoreType.DMA,
            pltpu.SemaphoreType.DMA,
            pltpu.SemaphoreType.DMA,
            pltpu.SemaphoreType.DMA,
        ],
        compiler_params=pltpu.CompilerParams(use_tc_tiling_on_sc=False,
                                             needs_layout_passes=False),
    )
    out = run(idx_t, a_t, t2f, p2, beta)
    return out.reshape(_BATCH, _SEQ, _D)
